# baseline (speedup 1.0000x reference)
import functools

import jax
import jax.numpy as jnp
from jax import lax
from jax.experimental import pallas as pl
from jax.experimental.pallas import tpu as pltpu
from jax.experimental.pallas import tpu_sc as plsc

N = 10000
E = 160000
H = 32
D_IN = 128
D_EDGE = 16
E_HID = 128
HH = H * H

NW = 32
CHUNK = 128
NCH = 40
EPT = NCH * CHUNK
EP = NW * EPT
NROW = 10240
SAC = N
NSUB = 16
RPT = NROW // NSUB

BE = 512
BN = 1000


def _sc_mesh():
    return plsc.VectorSubcoreMesh(core_axis_name="c", subcore_axis_name="s")


def _lin0_body(x_ref, w_ref, b_ref, o_ref):
    o_ref[...] = jnp.maximum(x_ref[...] @ w_ref[...] + b_ref[...], 0.0)


def _lin0(x, W0, b0):
    return pl.pallas_call(
        _lin0_body,
        grid=(N // BN,),
        in_specs=[
            pl.BlockSpec((BN, D_IN), lambda i: (i, 0)),
            pl.BlockSpec((D_IN, H), lambda i: (0, 0)),
            pl.BlockSpec((1, H), lambda i: (0, 0)),
        ],
        out_specs=pl.BlockSpec((BN, H), lambda i: (i, 0)),
        out_shape=jax.ShapeDtypeStruct((N, H), jnp.float32),
    )(x, W0, b0.reshape(1, H))


def _theta_body(ea_ref, w1_ref, b1_ref, w2_ref, b2_ref, o_ref):
    h = jnp.maximum(ea_ref[...] @ w1_ref[...] + b1_ref[...], 0.0)
    o_ref[...] = h @ w2_ref[...] + b2_ref[...]


def _theta(ea_p, We1, be1, We2, be2):
    return pl.pallas_call(
        _theta_body,
        grid=(EP // BE,),
        in_specs=[
            pl.BlockSpec((BE, D_EDGE), lambda i: (i, 0)),
            pl.BlockSpec((D_EDGE, E_HID), lambda i: (0, 0)),
            pl.BlockSpec((1, E_HID), lambda i: (0, 0)),
            pl.BlockSpec((E_HID, HH), lambda i: (0, 0)),
            pl.BlockSpec((1, HH), lambda i: (0, 0)),
        ],
        out_specs=pl.BlockSpec((BE, HH), lambda i: (i, 0)),
        out_shape=jax.ShapeDtypeStruct((EP, HH), jnp.float32),
    )(ea_p, We1, be1.reshape(1, E_HID), We2, be2.reshape(1, HH))


def _matvec_body(th_ref, g_ref, o_ref):
    g = g_ref[...]
    th = th_ref[...]
    acc = g[:, 0:1] * th[:, 0:H]
    for h in range(1, H):
        acc = acc + g[:, h:h + 1] * th[:, h * H:(h + 1) * H]
    o_ref[...] = acc


def _matvec(theta, gathered):
    return pl.pallas_call(
        _matvec_body,
        grid=(EP // BE,),
        in_specs=[
            pl.BlockSpec((BE, HH), lambda i: (i, 0)),
            pl.BlockSpec((BE, H), lambda i: (i, 0)),
        ],
        out_specs=pl.BlockSpec((BE, H), lambda i: (i, 0)),
        out_shape=jax.ShapeDtypeStruct((EP, H), jnp.float32),
    )(theta, gathered)


def _root_body(o_ref, p_ref, c_ref, w_ref, b_ref, out_ref):
    p = p_ref[...]
    cnt = c_ref[...]
    agg = p[0] + p[1]
    inv = 1.0 / jnp.maximum(cnt[0] + cnt[1], 1.0)
    out_ref[...] = jnp.maximum(
        o_ref[...] @ w_ref[...] + agg * inv + b_ref[...], 0.0)


def _root(out, parts, cnt, Wroot, broot):
    return pl.pallas_call(
        _root_body,
        grid=(N // BN,),
        in_specs=[
            pl.BlockSpec((BN, H), lambda i: (i, 0)),
            pl.BlockSpec((2, BN, H), lambda i: (0, i, 0)),
            pl.BlockSpec((2, BN, H), lambda i: (0, i, 0)),
            pl.BlockSpec((H, H), lambda i: (0, 0)),
            pl.BlockSpec((1, H), lambda i: (0, 0)),
        ],
        out_specs=pl.BlockSpec((BN, H), lambda i: (i, 0)),
        out_shape=jax.ShapeDtypeStruct((N, H), jnp.float32),
    )(out, parts, cnt, Wroot, broot.reshape(1, H))


def _head_body(o_ref, g_ref, be_ref, wy_ref, by_ref, y_ref):
    scale = 1.0 / jnp.sqrt(jnp.float32(1.0 + 1e-5))
    ybn = o_ref[...] * (g_ref[...] * scale) + be_ref[...]
    logits = ybn @ wy_ref[...] + by_ref[...]
    y_ref[...] = 1.0 / (1.0 + jnp.exp(-logits))


def _head(out, gamma, beta, Wy_p, by_p):
    return pl.pallas_call(
        _head_body,
        grid=(N // BN,),
        in_specs=[
            pl.BlockSpec((BN, H), lambda i: (i, 0)),
            pl.BlockSpec((1, H), lambda i: (0, 0)),
            pl.BlockSpec((1, H), lambda i: (0, 0)),
            pl.BlockSpec((H, 8), lambda i: (0, 0)),
            pl.BlockSpec((1, 8), lambda i: (0, 0)),
        ],
        out_specs=pl.BlockSpec((BN, 8), lambda i: (i, 0)),
        out_shape=jax.ShapeDtypeStruct((N, 8), jnp.float32),
    )(out, gamma.reshape(1, H), beta.reshape(1, H), Wy_p, by_p)


def _sc_gather(nodes, idx3):
    @functools.partial(
        pl.kernel,
        out_type=jax.ShapeDtypeStruct((EP, H), jnp.float32),
        mesh=_sc_mesh(),
        compiler_params=pltpu.CompilerParams(use_tc_tiling_on_sc=False),
        scratch_types=[
            pltpu.VMEM((NCH, CHUNK), jnp.int32),
            pltpu.VMEM((CHUNK, H), jnp.float32),
        ],
    )
    def k(nodes_hbm, idx_hbm, out_hbm, idx_v, buf_v):
        c = lax.axis_index("c")
        s = lax.axis_index("s")
        wid = s * 2 + c
        pltpu.sync_copy(idx_hbm.at[wid], idx_v)
        base = wid * EPT

        @pl.loop(0, NCH)
        def _(j):
            pltpu.sync_copy(nodes_hbm.at[idx_v.at[j]], buf_v)
            pltpu.sync_copy(buf_v, out_hbm.at[pl.ds(base + j * CHUNK, CHUNK)])

    return k(nodes, idx3)


def _sc_scatter(msg, idx3, zinit):
    @functools.partial(
        pl.kernel,
        out_type=jax.ShapeDtypeStruct((2, NROW, H), jnp.float32),
        mesh=_sc_mesh(),
        compiler_params=pltpu.CompilerParams(use_tc_tiling_on_sc=False),
        scratch_types=[
            pltpu.VMEM_SHARED((NROW, H), jnp.float32),
            pltpu.VMEM((NCH, CHUNK), jnp.int32),
            pltpu.VMEM((CHUNK, H), jnp.float32),
        ],
    )
    def k(msg_hbm, idx_hbm, z_hbm, out_hbm, aggr_sh, idx_v, buf_v):
        c = lax.axis_index("c")
        s = lax.axis_index("s")
        wid = s * 2 + c
        pltpu.sync_copy(z_hbm.at[pl.ds(s * RPT, RPT)],
                        aggr_sh.at[pl.ds(s * RPT, RPT)])
        pltpu.sync_copy(idx_hbm.at[wid], idx_v)
        plsc.subcore_barrier()
        base = wid * EPT

        @pl.loop(0, NCH)
        def _(j):
            pltpu.sync_copy(msg_hbm.at[pl.ds(base + j * CHUNK, CHUNK)], buf_v)
            pltpu.sync_copy(buf_v, aggr_sh.at[idx_v.at[j]], add=True)

        plsc.subcore_barrier()
        pltpu.sync_copy(aggr_sh.at[pl.ds(s * RPT, RPT)],
                        out_hbm.at[c, pl.ds(s * RPT, RPT)])

    return k(msg, idx3, zinit)


def _sc_count(idx3, ones, zinit):
    @functools.partial(
        pl.kernel,
        out_type=jax.ShapeDtypeStruct((2, NROW, H), jnp.float32),
        mesh=_sc_mesh(),
        compiler_params=pltpu.CompilerParams(use_tc_tiling_on_sc=False),
        scratch_types=[
            pltpu.VMEM_SHARED((NROW, H), jnp.float32),
            pltpu.VMEM((NCH, CHUNK), jnp.int32),
            pltpu.VMEM((CHUNK, H), jnp.float32),
        ],
    )
    def k(idx_hbm, ones_hbm, z_hbm, out_hbm, aggr_sh, idx_v, buf_v):
        c = lax.axis_index("c")
        s = lax.axis_index("s")
        wid = s * 2 + c
        pltpu.sync_copy(z_hbm.at[pl.ds(s * RPT, RPT)],
                        aggr_sh.at[pl.ds(s * RPT, RPT)])
        pltpu.sync_copy(ones_hbm, buf_v)
        pltpu.sync_copy(idx_hbm.at[wid], idx_v)
        plsc.subcore_barrier()

        @pl.loop(0, NCH)
        def _(j):
            pltpu.sync_copy(buf_v, aggr_sh.at[idx_v.at[j]], add=True)

        plsc.subcore_barrier()
        pltpu.sync_copy(aggr_sh.at[pl.ds(s * RPT, RPT)],
                        out_hbm.at[c, pl.ds(s * RPT, RPT)])

    return k(idx3, ones, zinit)


def kernel(x, edge_index, edge_attr, W0, b0, We1, be1, We2, be2,
           Wroot, broot, gamma, beta, Wy, by):
    pad = EP - E
    src_p = jnp.concatenate(
        [edge_index[0], jnp.zeros((pad,), jnp.int32)]).reshape(NW, NCH, CHUNK)
    dst_p = jnp.concatenate(
        [edge_index[1], jnp.full((pad,), SAC, jnp.int32)]).reshape(NW, NCH, CHUNK)
    ea_p = jnp.concatenate(
        [edge_attr, jnp.zeros((pad, D_EDGE), jnp.float32)])
    zinit = jnp.zeros((NROW, H), jnp.float32)
    ones = jnp.ones((CHUNK, H), jnp.float32)
    Wy_p = jnp.concatenate([Wy, jnp.zeros((H, 8 - Wy.shape[1]), jnp.float32)], axis=1)
    by_p = jnp.concatenate([by, jnp.zeros((8 - by.shape[0],), jnp.float32)]).reshape(1, 8)

    out = _lin0(x, W0, b0)
    theta = _theta(ea_p, We1, be1, We2, be2)
    cnt = _sc_count(dst_p, ones, zinit)
    for _ in range(3):
        gathered = _sc_gather(out, src_p)
        msg = _matvec(theta, gathered)
        parts = _sc_scatter(msg, dst_p, zinit)
        out = _root(out, parts, cnt, Wroot, broot)
    y = _head(out, gamma, beta, Wy_p, by_p)
    return y[:, :2]


# speedup vs baseline: 2.2146x; 2.2146x over previous
import functools

import jax
import jax.numpy as jnp
from jax import lax
from jax.experimental import pallas as pl
from jax.experimental.pallas import tpu as pltpu
from jax.experimental.pallas import tpu_sc as plsc

N = 10000
E = 160000
H = 32
D_IN = 128
D_EDGE = 16
E_HID = 128
HH = H * H

NW = 32
CHUNK = 128
NCH = 40
EPT = NCH * CHUNK
EP = NW * EPT
NROW = 10240
SAC = N
NSUB = 16
RPT = NROW // NSUB

BE = 512
BN = 1000


def _sc_mesh():
    return plsc.VectorSubcoreMesh(core_axis_name="c", subcore_axis_name="s")


def _lin0_body(x_ref, w_ref, b_ref, o_ref):
    o_ref[...] = jnp.maximum(x_ref[...] @ w_ref[...] + b_ref[...], 0.0)


def _lin0(x, W0, b0):
    return pl.pallas_call(
        _lin0_body,
        grid=(N // BN,),
        in_specs=[
            pl.BlockSpec((BN, D_IN), lambda i: (i, 0)),
            pl.BlockSpec((D_IN, H), lambda i: (0, 0)),
            pl.BlockSpec((1, H), lambda i: (0, 0)),
        ],
        out_specs=pl.BlockSpec((BN, H), lambda i: (i, 0)),
        out_shape=jax.ShapeDtypeStruct((N, H), jnp.float32),
    )(x, W0, b0.reshape(1, H))


def _theta_body(ea_ref, w1_ref, b1_ref, w2_ref, b2_ref, o_ref):
    h = jnp.maximum(ea_ref[...] @ w1_ref[...] + b1_ref[...], 0.0)
    o_ref[...] = h @ w2_ref[...] + b2_ref[...]


def _theta(ea_p, We1, be1, We2, be2):
    return pl.pallas_call(
        _theta_body,
        grid=(EP // BE,),
        in_specs=[
            pl.BlockSpec((BE, D_EDGE), lambda i: (i, 0)),
            pl.BlockSpec((D_EDGE, E_HID), lambda i: (0, 0)),
            pl.BlockSpec((1, E_HID), lambda i: (0, 0)),
            pl.BlockSpec((E_HID, HH), lambda i: (0, 0)),
            pl.BlockSpec((1, HH), lambda i: (0, 0)),
        ],
        out_specs=pl.BlockSpec((BE, HH), lambda i: (i, 0)),
        out_shape=jax.ShapeDtypeStruct((EP, HH), jnp.float32),
    )(ea_p, We1, be1.reshape(1, E_HID), We2, be2.reshape(1, HH))


def _matvec_body(th_ref, g_ref, r_ref, o_ref):
    grep = jnp.dot(g_ref[...], r_ref[...], preferred_element_type=jnp.float32)
    prod = grep * th_ref[...]
    acc = prod[:, 0:128]
    for q in range(1, 8):
        acc = acc + prod[:, 128 * q:128 * (q + 1)]
    m64 = acc[:, 0:64] + acc[:, 64:128]
    o_ref[...] = m64[:, 0:H] + m64[:, H:2 * H]


def _matvec(theta, gathered, R):
    return pl.pallas_call(
        _matvec_body,
        grid=(EP // BE,),
        in_specs=[
            pl.BlockSpec((BE, HH), lambda i: (i, 0)),
            pl.BlockSpec((BE, H), lambda i: (i, 0)),
            pl.BlockSpec((H, HH), lambda i: (0, 0)),
        ],
        out_specs=pl.BlockSpec((BE, H), lambda i: (i, 0)),
        out_shape=jax.ShapeDtypeStruct((EP, H), jnp.float32),
    )(theta, gathered, R)


def _root_body(o_ref, p_ref, c_ref, w_ref, b_ref, out_ref):
    p = p_ref[...]
    cnt = c_ref[...]
    agg = p[0] + p[1]
    inv = 1.0 / jnp.maximum(cnt[0] + cnt[1], 1.0)
    out_ref[...] = jnp.maximum(
        o_ref[...] @ w_ref[...] + agg * inv + b_ref[...], 0.0)


def _root(out, parts, cnt, Wroot, broot):
    return pl.pallas_call(
        _root_body,
        grid=(N // BN,),
        in_specs=[
            pl.BlockSpec((BN, H), lambda i: (i, 0)),
            pl.BlockSpec((2, BN, H), lambda i: (0, i, 0)),
            pl.BlockSpec((2, BN, H), lambda i: (0, i, 0)),
            pl.BlockSpec((H, H), lambda i: (0, 0)),
            pl.BlockSpec((1, H), lambda i: (0, 0)),
        ],
        out_specs=pl.BlockSpec((BN, H), lambda i: (i, 0)),
        out_shape=jax.ShapeDtypeStruct((N, H), jnp.float32),
    )(out, parts, cnt, Wroot, broot.reshape(1, H))


def _head_body(o_ref, g_ref, be_ref, wy_ref, by_ref, y_ref):
    scale = 1.0 / jnp.sqrt(jnp.float32(1.0 + 1e-5))
    ybn = o_ref[...] * (g_ref[...] * scale) + be_ref[...]
    logits = ybn @ wy_ref[...] + by_ref[...]
    y_ref[...] = 1.0 / (1.0 + jnp.exp(-logits))


def _head(out, gamma, beta, Wy_p, by_p):
    return pl.pallas_call(
        _head_body,
        grid=(N // BN,),
        in_specs=[
            pl.BlockSpec((BN, H), lambda i: (i, 0)),
            pl.BlockSpec((1, H), lambda i: (0, 0)),
            pl.BlockSpec((1, H), lambda i: (0, 0)),
            pl.BlockSpec((H, 8), lambda i: (0, 0)),
            pl.BlockSpec((1, 8), lambda i: (0, 0)),
        ],
        out_specs=pl.BlockSpec((BN, 8), lambda i: (i, 0)),
        out_shape=jax.ShapeDtypeStruct((N, 8), jnp.float32),
    )(out, gamma.reshape(1, H), beta.reshape(1, H), Wy_p, by_p)


def _sc_gather(nodes, idx3):
    @functools.partial(
        pl.kernel,
        out_type=jax.ShapeDtypeStruct((EP, H), jnp.float32),
        mesh=_sc_mesh(),
        compiler_params=pltpu.CompilerParams(use_tc_tiling_on_sc=False),
        scratch_types=[
            pltpu.VMEM((NCH, CHUNK), jnp.int32),
            pltpu.VMEM((CHUNK, H), jnp.float32),
        ],
    )
    def k(nodes_hbm, idx_hbm, out_hbm, idx_v, buf_v):
        c = lax.axis_index("c")
        s = lax.axis_index("s")
        wid = s * 2 + c
        pltpu.sync_copy(idx_hbm.at[wid], idx_v)
        base = wid * EPT

        @pl.loop(0, NCH)
        def _(j):
            pltpu.sync_copy(nodes_hbm.at[idx_v.at[j]], buf_v)
            pltpu.sync_copy(buf_v, out_hbm.at[pl.ds(base + j * CHUNK, CHUNK)])

    return k(nodes, idx3)


def _sc_scatter(msg, idx3, zinit):
    @functools.partial(
        pl.kernel,
        out_type=jax.ShapeDtypeStruct((2, NROW, H), jnp.float32),
        mesh=_sc_mesh(),
        compiler_params=pltpu.CompilerParams(use_tc_tiling_on_sc=False),
        scratch_types=[
            pltpu.VMEM_SHARED((NROW, H), jnp.float32),
            pltpu.VMEM((NCH, CHUNK), jnp.int32),
            pltpu.VMEM((CHUNK, H), jnp.float32),
        ],
    )
    def k(msg_hbm, idx_hbm, z_hbm, out_hbm, aggr_sh, idx_v, buf_v):
        c = lax.axis_index("c")
        s = lax.axis_index("s")
        wid = s * 2 + c
        pltpu.sync_copy(z_hbm.at[pl.ds(s * RPT, RPT)],
                        aggr_sh.at[pl.ds(s * RPT, RPT)])
        pltpu.sync_copy(idx_hbm.at[wid], idx_v)
        plsc.subcore_barrier()
        base = wid * EPT

        @pl.loop(0, NCH)
        def _(j):
            pltpu.sync_copy(msg_hbm.at[pl.ds(base + j * CHUNK, CHUNK)], buf_v)
            pltpu.sync_copy(buf_v, aggr_sh.at[idx_v.at[j]], add=True)

        plsc.subcore_barrier()
        pltpu.sync_copy(aggr_sh.at[pl.ds(s * RPT, RPT)],
                        out_hbm.at[c, pl.ds(s * RPT, RPT)])

    return k(msg, idx3, zinit)


def _sc_count(idx3, ones, zinit):
    @functools.partial(
        pl.kernel,
        out_type=jax.ShapeDtypeStruct((2, NROW, H), jnp.float32),
        mesh=_sc_mesh(),
        compiler_params=pltpu.CompilerParams(use_tc_tiling_on_sc=False),
        scratch_types=[
            pltpu.VMEM_SHARED((NROW, H), jnp.float32),
            pltpu.VMEM((NCH, CHUNK), jnp.int32),
            pltpu.VMEM((CHUNK, H), jnp.float32),
        ],
    )
    def k(idx_hbm, ones_hbm, z_hbm, out_hbm, aggr_sh, idx_v, buf_v):
        c = lax.axis_index("c")
        s = lax.axis_index("s")
        wid = s * 2 + c
        pltpu.sync_copy(z_hbm.at[pl.ds(s * RPT, RPT)],
                        aggr_sh.at[pl.ds(s * RPT, RPT)])
        pltpu.sync_copy(ones_hbm, buf_v)
        pltpu.sync_copy(idx_hbm.at[wid], idx_v)
        plsc.subcore_barrier()

        @pl.loop(0, NCH)
        def _(j):
            pltpu.sync_copy(buf_v, aggr_sh.at[idx_v.at[j]], add=True)

        plsc.subcore_barrier()
        pltpu.sync_copy(aggr_sh.at[pl.ds(s * RPT, RPT)],
                        out_hbm.at[c, pl.ds(s * RPT, RPT)])

    return k(idx3, ones, zinit)


def kernel(x, edge_index, edge_attr, W0, b0, We1, be1, We2, be2,
           Wroot, broot, gamma, beta, Wy, by):
    pad = EP - E
    src_p = jnp.concatenate(
        [edge_index[0], jnp.zeros((pad,), jnp.int32)]).reshape(NW, NCH, CHUNK)
    dst_p = jnp.concatenate(
        [edge_index[1], jnp.full((pad,), SAC, jnp.int32)]).reshape(NW, NCH, CHUNK)
    ea_p = jnp.concatenate(
        [edge_attr, jnp.zeros((pad, D_EDGE), jnp.float32)])
    zinit = jnp.zeros((NROW, H), jnp.float32)
    ones = jnp.ones((CHUNK, H), jnp.float32)
    lane = jnp.arange(HH, dtype=jnp.int32)
    h_of_lane = 4 * (lane // 128) + (lane % 128) // H
    R = (h_of_lane[None, :] == jnp.arange(H, dtype=jnp.int32)[:, None]
         ).astype(jnp.float32)
    Wy_p = jnp.concatenate([Wy, jnp.zeros((H, 8 - Wy.shape[1]), jnp.float32)], axis=1)
    by_p = jnp.concatenate([by, jnp.zeros((8 - by.shape[0],), jnp.float32)]).reshape(1, 8)

    out = _lin0(x, W0, b0)
    theta = _theta(ea_p, We1, be1, We2, be2)
    cnt = _sc_count(dst_p, ones, zinit)
    for _ in range(3):
        gathered = _sc_gather(out, src_p)
        msg = _matvec(theta, gathered, R)
        parts = _sc_scatter(msg, dst_p, zinit)
        out = _root(out, parts, cnt, Wroot, broot)
    y = _head(out, gamma, beta, Wy_p, by_p)
    return y[:, :2]


# speedup vs baseline: 2.2365x; 1.0099x over previous
import functools

import jax
import jax.numpy as jnp
from jax import lax
from jax.experimental import pallas as pl
from jax.experimental.pallas import tpu as pltpu
from jax.experimental.pallas import tpu_sc as plsc

N = 10000
E = 160000
H = 32
D_IN = 128
D_EDGE = 16
E_HID = 128
HH = H * H
W = 128

NW = 32
CHUNK = 128
NCH = 40
EPT = NCH * CHUNK
EP = NW * EPT
NROW = 10240
SAC = N
NSUB = 16
RPT = NROW // NSUB

BE = 512
BN = 1000


def _sc_mesh():
    return plsc.VectorSubcoreMesh(core_axis_name="c", subcore_axis_name="s")


def _lin0_body(x_ref, w_ref, b_ref, o_ref):
    o_ref[...] = jnp.maximum(x_ref[...] @ w_ref[...] + b_ref[...], 0.0)


def _lin0(x, W0p, b0p):
    return pl.pallas_call(
        _lin0_body,
        grid=(N // BN,),
        in_specs=[
            pl.BlockSpec((BN, D_IN), lambda i: (i, 0)),
            pl.BlockSpec((D_IN, W), lambda i: (0, 0)),
            pl.BlockSpec((1, W), lambda i: (0, 0)),
        ],
        out_specs=pl.BlockSpec((BN, W), lambda i: (i, 0)),
        out_shape=jax.ShapeDtypeStruct((N, W), jnp.float32),
    )(x, W0p, b0p)


def _theta_body(ea_ref, w1_ref, b1_ref, w2_ref, b2_ref, o_ref):
    h = jnp.maximum(ea_ref[...] @ w1_ref[...] + b1_ref[...], 0.0)
    o_ref[...] = (h @ w2_ref[...] + b2_ref[...]).astype(jnp.bfloat16)


def _theta(ea_p, We1, be1, We2, be2):
    return pl.pallas_call(
        _theta_body,
        grid=(EP // BE,),
        in_specs=[
            pl.BlockSpec((BE, D_EDGE), lambda i: (i, 0)),
            pl.BlockSpec((D_EDGE, E_HID), lambda i: (0, 0)),
            pl.BlockSpec((1, E_HID), lambda i: (0, 0)),
            pl.BlockSpec((E_HID, HH), lambda i: (0, 0)),
            pl.BlockSpec((1, HH), lambda i: (0, 0)),
        ],
        out_specs=pl.BlockSpec((BE, HH), lambda i: (i, 0)),
        out_shape=jax.ShapeDtypeStruct((EP, HH), jnp.bfloat16),
    )(ea_p, We1, be1.reshape(1, E_HID), We2, be2.reshape(1, HH))


def _matvec_body(th_ref, g_ref, r_ref, o_ref):
    grep = jnp.dot(g_ref[...], r_ref[...], preferred_element_type=jnp.float32)
    prod = grep * th_ref[...].astype(jnp.float32)
    acc = prod[:, 0:128]
    for q in range(1, 8):
        acc = acc + prod[:, 128 * q:128 * (q + 1)]
    m64 = acc[:, 0:64] + acc[:, 64:128]
    m32 = m64[:, 0:H] + m64[:, H:2 * H]
    o_ref[...] = jnp.concatenate(
        [m32, jnp.zeros((m32.shape[0], W - H), jnp.float32)], axis=1)


def _matvec(theta, gathered, R):
    return pl.pallas_call(
        _matvec_body,
        grid=(EP // BE,),
        in_specs=[
            pl.BlockSpec((BE, HH), lambda i: (i, 0)),
            pl.BlockSpec((BE, W), lambda i: (i, 0)),
            pl.BlockSpec((W, HH), lambda i: (0, 0)),
        ],
        out_specs=pl.BlockSpec((BE, W), lambda i: (i, 0)),
        out_shape=jax.ShapeDtypeStruct((EP, W), jnp.float32),
    )(theta, gathered, R)


def _root_body(o_ref, p_ref, c_ref, w_ref, b_ref, out_ref):
    p = p_ref[...]
    cnt = c_ref[...]
    agg = p[0] + p[1]
    inv = 1.0 / jnp.maximum(cnt[0] + cnt[1], 1.0)
    out_ref[...] = jnp.maximum(
        o_ref[...] @ w_ref[...] + agg * inv + b_ref[...], 0.0)


def _root(out, parts, cnt, Wrootp, brootp):
    return pl.pallas_call(
        _root_body,
        grid=(N // BN,),
        in_specs=[
            pl.BlockSpec((BN, W), lambda i: (i, 0)),
            pl.BlockSpec((2, BN, W), lambda i: (0, i, 0)),
            pl.BlockSpec((2, BN, W), lambda i: (0, i, 0)),
            pl.BlockSpec((W, W), lambda i: (0, 0)),
            pl.BlockSpec((1, W), lambda i: (0, 0)),
        ],
        out_specs=pl.BlockSpec((BN, W), lambda i: (i, 0)),
        out_shape=jax.ShapeDtypeStruct((N, W), jnp.float32),
    )(out, parts, cnt, Wrootp, brootp)


def _head_body(o_ref, g_ref, be_ref, wy_ref, by_ref, y_ref):
    scale = 1.0 / jnp.sqrt(jnp.float32(1.0 + 1e-5))
    ybn = o_ref[...] * (g_ref[...] * scale) + be_ref[...]
    logits = ybn @ wy_ref[...] + by_ref[...]
    y_ref[...] = 1.0 / (1.0 + jnp.exp(-logits))


def _head(out, gammap, betap, Wy_p, by_p):
    return pl.pallas_call(
        _head_body,
        grid=(N // BN,),
        in_specs=[
            pl.BlockSpec((BN, W), lambda i: (i, 0)),
            pl.BlockSpec((1, W), lambda i: (0, 0)),
            pl.BlockSpec((1, W), lambda i: (0, 0)),
            pl.BlockSpec((W, 8), lambda i: (0, 0)),
            pl.BlockSpec((1, 8), lambda i: (0, 0)),
        ],
        out_specs=pl.BlockSpec((BN, 8), lambda i: (i, 0)),
        out_shape=jax.ShapeDtypeStruct((N, 8), jnp.float32),
    )(out, gammap, betap, Wy_p, by_p)


def _sc_gather(nodes, idx3):
    @functools.partial(
        pl.kernel,
        out_type=jax.ShapeDtypeStruct((EP, W), jnp.float32),
        mesh=_sc_mesh(),
        scratch_types=[
            pltpu.VMEM((NCH, CHUNK), jnp.int32),
            pltpu.VMEM((CHUNK, W), jnp.float32),
            pltpu.VMEM((CHUNK, W), jnp.float32),
            pltpu.SemaphoreType.DMA,
            pltpu.SemaphoreType.DMA,
        ],
    )
    def k(nodes_hbm, idx_hbm, out_hbm, idx_v, buf_a, buf_b, sem_a, sem_b):
        c = lax.axis_index("c")
        s = lax.axis_index("s")
        wid = s * 2 + c
        pltpu.sync_copy(idx_hbm.at[wid], idx_v)
        base = wid * EPT

        @pl.loop(0, NCH, step=2)
        def _(j):
            da = pltpu.async_copy(nodes_hbm.at[idx_v.at[j]], buf_a, sem_a)
            db = pltpu.async_copy(nodes_hbm.at[idx_v.at[j + 1]], buf_b, sem_b)
            da.wait()
            pltpu.sync_copy(buf_a, out_hbm.at[pl.ds(base + j * CHUNK, CHUNK)])
            db.wait()
            pltpu.sync_copy(buf_b,
                            out_hbm.at[pl.ds(base + (j + 1) * CHUNK, CHUNK)])

    return k(nodes, idx3)


def _sc_scatter(msg, idx3, zinit):
    @functools.partial(
        pl.kernel,
        out_type=jax.ShapeDtypeStruct((2, NROW, W), jnp.float32),
        mesh=_sc_mesh(),
        scratch_types=[
            pltpu.VMEM_SHARED((NROW, W), jnp.float32),
            pltpu.VMEM((NCH, CHUNK), jnp.int32),
            pltpu.VMEM((CHUNK, W), jnp.float32),
            pltpu.VMEM((CHUNK, W), jnp.float32),
            pltpu.SemaphoreType.DMA,
            pltpu.SemaphoreType.DMA,
        ],
    )
    def k(msg_hbm, idx_hbm, z_hbm, out_hbm, aggr_sh, idx_v, buf_a, buf_b,
          sem_a, sem_b):
        c = lax.axis_index("c")
        s = lax.axis_index("s")
        wid = s * 2 + c
        pltpu.sync_copy(z_hbm.at[pl.ds(s * RPT, RPT)],
                        aggr_sh.at[pl.ds(s * RPT, RPT)])
        pltpu.sync_copy(idx_hbm.at[wid], idx_v)
        plsc.subcore_barrier()
        base = wid * EPT

        @pl.loop(0, NCH, step=2)
        def _(j):
            da = pltpu.async_copy(
                msg_hbm.at[pl.ds(base + j * CHUNK, CHUNK)], buf_a, sem_a)
            db = pltpu.async_copy(
                msg_hbm.at[pl.ds(base + (j + 1) * CHUNK, CHUNK)], buf_b, sem_b)
            da.wait()
            pltpu.sync_copy(buf_a, aggr_sh.at[idx_v.at[j]], add=True)
            db.wait()
            pltpu.sync_copy(buf_b, aggr_sh.at[idx_v.at[j + 1]], add=True)

        plsc.subcore_barrier()
        pltpu.sync_copy(aggr_sh.at[pl.ds(s * RPT, RPT)],
                        out_hbm.at[c, pl.ds(s * RPT, RPT)])

    return k(msg, idx3, zinit)


def _sc_count(idx3, ones, zinit):
    @functools.partial(
        pl.kernel,
        out_type=jax.ShapeDtypeStruct((2, NROW, W), jnp.float32),
        mesh=_sc_mesh(),
        scratch_types=[
            pltpu.VMEM_SHARED((NROW, W), jnp.float32),
            pltpu.VMEM((NCH, CHUNK), jnp.int32),
            pltpu.VMEM((CHUNK, W), jnp.float32),
        ],
    )
    def k(idx_hbm, ones_hbm, z_hbm, out_hbm, aggr_sh, idx_v, buf_v):
        c = lax.axis_index("c")
        s = lax.axis_index("s")
        wid = s * 2 + c
        pltpu.sync_copy(z_hbm.at[pl.ds(s * RPT, RPT)],
                        aggr_sh.at[pl.ds(s * RPT, RPT)])
        pltpu.sync_copy(ones_hbm, buf_v)
        pltpu.sync_copy(idx_hbm.at[wid], idx_v)
        plsc.subcore_barrier()

        @pl.loop(0, NCH)
        def _(j):
            pltpu.sync_copy(buf_v, aggr_sh.at[idx_v.at[j]], add=True)

        plsc.subcore_barrier()
        pltpu.sync_copy(aggr_sh.at[pl.ds(s * RPT, RPT)],
                        out_hbm.at[c, pl.ds(s * RPT, RPT)])

    return k(idx3, ones, zinit)


def kernel(x, edge_index, edge_attr, W0, b0, We1, be1, We2, be2,
           Wroot, broot, gamma, beta, Wy, by):
    pad = EP - E
    src_p = jnp.concatenate(
        [edge_index[0], jnp.zeros((pad,), jnp.int32)]).reshape(NW, NCH, CHUNK)
    dst_p = jnp.concatenate(
        [edge_index[1], jnp.full((pad,), SAC, jnp.int32)]).reshape(NW, NCH, CHUNK)
    ea_p = jnp.concatenate(
        [edge_attr, jnp.zeros((pad, D_EDGE), jnp.float32)])
    zinit = jnp.zeros((NROW, W), jnp.float32)
    ones = jnp.ones((CHUNK, W), jnp.float32)

    lane = jnp.arange(HH, dtype=jnp.int32)
    h_of_lane = 4 * (lane // 128) + (lane % 128) // H
    R = (h_of_lane[None, :] == jnp.arange(W, dtype=jnp.int32)[:, None]
         ).astype(jnp.float32)

    zc = jnp.zeros((D_IN, W - H), jnp.float32)
    W0p = jnp.concatenate([W0, zc], axis=1)
    b0p = jnp.concatenate([b0, jnp.zeros((W - H,), jnp.float32)]).reshape(1, W)
    Wrootp = jnp.zeros((W, W), jnp.float32).at[:H, :H].set(Wroot)
    brootp = jnp.concatenate(
        [broot, jnp.zeros((W - H,), jnp.float32)]).reshape(1, W)
    gammap = jnp.concatenate(
        [gamma, jnp.zeros((W - H,), jnp.float32)]).reshape(1, W)
    betap = jnp.concatenate(
        [beta, jnp.zeros((W - H,), jnp.float32)]).reshape(1, W)
    Wy_p = jnp.zeros((W, 8), jnp.float32).at[:H, :Wy.shape[1]].set(Wy)
    by_p = jnp.concatenate(
        [by, jnp.zeros((8 - by.shape[0],), jnp.float32)]).reshape(1, 8)

    out = _lin0(x, W0p, b0p)
    theta = _theta(ea_p, We1, be1, We2, be2)
    cnt = _sc_count(dst_p, ones, zinit)
    for _ in range(3):
        gathered = _sc_gather(out, src_p)
        msg = _matvec(theta, gathered, R)
        parts = _sc_scatter(msg, dst_p, zinit)
        out = _root(out, parts, cnt, Wrootp, brootp)
    y = _head(out, gammap, betap, Wy_p, by_p)
    return y[:, :2]


# speedup vs baseline: 2.2866x; 1.0224x over previous
import functools

import jax
import jax.numpy as jnp
from jax import lax
from jax.experimental import pallas as pl
from jax.experimental.pallas import tpu as pltpu
from jax.experimental.pallas import tpu_sc as plsc

N = 10000
E = 160000
H = 32
D_IN = 128
D_EDGE = 16
E_HID = 128
HH = H * H
W = 128

NW = 32
CHUNK = 128
NCH = 40
EPT = NCH * CHUNK
EP = NW * EPT
NROW = 10240
SAC = N
NSUB = 16
RPT = NROW // NSUB

BE = 512
BN = 1000


def _sc_mesh():
    return plsc.VectorSubcoreMesh(core_axis_name="c", subcore_axis_name="s")


def _lin0_body(x_ref, w_ref, b_ref, o_ref):
    o_ref[...] = jnp.maximum(x_ref[...] @ w_ref[...] + b_ref[...], 0.0)


def _lin0(x, W0p, b0p):
    return pl.pallas_call(
        _lin0_body,
        grid=(N // BN,),
        in_specs=[
            pl.BlockSpec((BN, D_IN), lambda i: (i, 0)),
            pl.BlockSpec((D_IN, W), lambda i: (0, 0)),
            pl.BlockSpec((1, W), lambda i: (0, 0)),
        ],
        out_specs=pl.BlockSpec((BN, W), lambda i: (i, 0)),
        out_shape=jax.ShapeDtypeStruct((N, W), jnp.float32),
    )(x, W0p, b0p)


def _theta_body(ea_ref, w1_ref, b1_ref, w2_ref, b2_ref, o_ref):
    h = jnp.maximum(ea_ref[...] @ w1_ref[...] + b1_ref[...], 0.0)
    hb = h.astype(jnp.bfloat16)
    th = jnp.dot(hb, w2_ref[...], preferred_element_type=jnp.float32)
    o_ref[...] = (th + b2_ref[...]).astype(jnp.bfloat16)


def _theta(ea, We1, be1, We2b, be2):
    return pl.pallas_call(
        _theta_body,
        grid=((E + BE - 1) // BE,),
        in_specs=[
            pl.BlockSpec((BE, D_EDGE), lambda i: (i, 0)),
            pl.BlockSpec((D_EDGE, E_HID), lambda i: (0, 0)),
            pl.BlockSpec((1, E_HID), lambda i: (0, 0)),
            pl.BlockSpec((E_HID, HH), lambda i: (0, 0)),
            pl.BlockSpec((1, HH), lambda i: (0, 0)),
        ],
        out_specs=pl.BlockSpec((BE, HH), lambda i: (i, 0)),
        out_shape=jax.ShapeDtypeStruct((EP, HH), jnp.bfloat16),
    )(ea, We1, be1.reshape(1, E_HID), We2b, be2.reshape(1, HH))


def _matvec_body(th_ref, g_ref, r_ref, o_ref):
    grep = jnp.dot(g_ref[...], r_ref[...], preferred_element_type=jnp.float32)
    prod = grep * th_ref[...].astype(jnp.float32)
    acc = prod[:, 0:128]
    for q in range(1, 8):
        acc = acc + prod[:, 128 * q:128 * (q + 1)]
    m64 = acc[:, 0:64] + acc[:, 64:128]
    m32 = m64[:, 0:H] + m64[:, H:2 * H]
    o_ref[...] = jnp.concatenate(
        [m32, jnp.zeros((m32.shape[0], W - H), jnp.float32)], axis=1)


def _matvec(theta, gathered, R):
    return pl.pallas_call(
        _matvec_body,
        grid=(EP // BE,),
        in_specs=[
            pl.BlockSpec((BE, HH), lambda i: (i, 0)),
            pl.BlockSpec((BE, W), lambda i: (i, 0)),
            pl.BlockSpec((W, HH), lambda i: (0, 0)),
        ],
        out_specs=pl.BlockSpec((BE, W), lambda i: (i, 0)),
        out_shape=jax.ShapeDtypeStruct((EP, W), jnp.float32),
    )(theta, gathered, R)


def _root_body(o_ref, p_ref, c_ref, w_ref, b_ref, out_ref):
    p = p_ref[...]
    cnt = c_ref[...]
    agg = p[0] + p[1]
    inv = 1.0 / jnp.maximum(cnt[0] + cnt[1], 1.0)
    aggw = jnp.concatenate(
        [agg * inv, jnp.zeros((agg.shape[0], W - H), jnp.float32)], axis=1)
    out_ref[...] = jnp.maximum(
        o_ref[...] @ w_ref[...] + aggw + b_ref[...], 0.0)


def _root(out, parts, cnt, Wrootp, brootp):
    return pl.pallas_call(
        _root_body,
        grid=(N // BN,),
        in_specs=[
            pl.BlockSpec((BN, W), lambda i: (i, 0)),
            pl.BlockSpec((2, BN, H), lambda i: (0, i, 0)),
            pl.BlockSpec((2, BN, H), lambda i: (0, i, 0)),
            pl.BlockSpec((W, W), lambda i: (0, 0)),
            pl.BlockSpec((1, W), lambda i: (0, 0)),
        ],
        out_specs=pl.BlockSpec((BN, W), lambda i: (i, 0)),
        out_shape=jax.ShapeDtypeStruct((N, W), jnp.float32),
    )(out, parts, cnt, Wrootp, brootp)


def _head_body(o_ref, g_ref, be_ref, wy_ref, by_ref, y_ref):
    scale = 1.0 / jnp.sqrt(jnp.float32(1.0 + 1e-5))
    ybn = o_ref[...] * (g_ref[...] * scale) + be_ref[...]
    logits = ybn @ wy_ref[...] + by_ref[...]
    y_ref[...] = 1.0 / (1.0 + jnp.exp(-logits))


def _head(out, gammap, betap, Wy_p, by_p):
    return pl.pallas_call(
        _head_body,
        grid=(N // BN,),
        in_specs=[
            pl.BlockSpec((BN, W), lambda i: (i, 0)),
            pl.BlockSpec((1, W), lambda i: (0, 0)),
            pl.BlockSpec((1, W), lambda i: (0, 0)),
            pl.BlockSpec((W, 8), lambda i: (0, 0)),
            pl.BlockSpec((1, 8), lambda i: (0, 0)),
        ],
        out_specs=pl.BlockSpec((BN, 8), lambda i: (i, 0)),
        out_shape=jax.ShapeDtypeStruct((N, 8), jnp.float32),
    )(out, gammap, betap, Wy_p, by_p)


def _sc_gather(nodes, idx3):
    @functools.partial(
        pl.kernel,
        out_type=jax.ShapeDtypeStruct((EP, W), jnp.float32),
        mesh=_sc_mesh(),
        scratch_types=[
            pltpu.VMEM((NCH, CHUNK), jnp.int32),
            pltpu.VMEM((CHUNK, W), jnp.float32),
            pltpu.VMEM((CHUNK, W), jnp.float32),
            pltpu.SemaphoreType.DMA,
            pltpu.SemaphoreType.DMA,
        ],
    )
    def k(nodes_hbm, idx_hbm, out_hbm, idx_v, buf_a, buf_b, sem_a, sem_b):
        c = lax.axis_index("c")
        s = lax.axis_index("s")
        wid = s * 2 + c
        pltpu.sync_copy(idx_hbm.at[wid], idx_v)
        base = wid * EPT

        @pl.loop(0, NCH, step=2)
        def _(j):
            da = pltpu.async_copy(nodes_hbm.at[idx_v.at[j]], buf_a, sem_a)
            db = pltpu.async_copy(nodes_hbm.at[idx_v.at[j + 1]], buf_b, sem_b)
            da.wait()
            pltpu.sync_copy(buf_a, out_hbm.at[pl.ds(base + j * CHUNK, CHUNK)])
            db.wait()
            pltpu.sync_copy(buf_b,
                            out_hbm.at[pl.ds(base + (j + 1) * CHUNK, CHUNK)])

    return k(nodes, idx3)


def _sc_scatter(msg, idx3, zinit):
    @functools.partial(
        pl.kernel,
        out_type=jax.ShapeDtypeStruct((2, NROW, H), jnp.float32),
        mesh=_sc_mesh(),
        compiler_params=pltpu.CompilerParams(use_tc_tiling_on_sc=False),
        scratch_types=[
            pltpu.VMEM_SHARED((NROW, H), jnp.float32),
            pltpu.VMEM((NCH, CHUNK), jnp.int32),
            pltpu.VMEM((CHUNK, W), jnp.float32),
            pltpu.VMEM((CHUNK, W), jnp.float32),
            pltpu.VMEM((CHUNK, H), jnp.float32),
            pltpu.VMEM((CHUNK, H), jnp.float32),
            pltpu.SemaphoreType.DMA,
            pltpu.SemaphoreType.DMA,
        ],
    )
    def k(msg_hbm, idx_hbm, z_hbm, out_hbm, aggr_sh, idx_v, buf_a, buf_b,
          nar_a, nar_b, sem_a, sem_b):
        c = lax.axis_index("c")
        s = lax.axis_index("s")
        wid = s * 2 + c
        pltpu.sync_copy(z_hbm.at[pl.ds(s * RPT, RPT)],
                        aggr_sh.at[pl.ds(s * RPT, RPT)])
        pltpu.sync_copy(idx_hbm.at[wid], idx_v)
        plsc.subcore_barrier()
        base = wid * EPT

        @pl.loop(0, NCH, step=2)
        def _(j):
            da = pltpu.async_copy(
                msg_hbm.at[pl.ds(base + j * CHUNK, CHUNK)], buf_a, sem_a)
            db = pltpu.async_copy(
                msg_hbm.at[pl.ds(base + (j + 1) * CHUNK, CHUNK)], buf_b, sem_b)
            da.wait()
            for r in range(CHUNK):
                nar_a[r, pl.ds(0, 16)] = buf_a[r, pl.ds(0, 16)]
                nar_a[r, pl.ds(16, 16)] = buf_a[r, pl.ds(16, 16)]
            pltpu.sync_copy(nar_a, aggr_sh.at[idx_v.at[j]], add=True)
            db.wait()
            for r in range(CHUNK):
                nar_b[r, pl.ds(0, 16)] = buf_b[r, pl.ds(0, 16)]
                nar_b[r, pl.ds(16, 16)] = buf_b[r, pl.ds(16, 16)]
            pltpu.sync_copy(nar_b, aggr_sh.at[idx_v.at[j + 1]], add=True)

        plsc.subcore_barrier()
        pltpu.sync_copy(aggr_sh.at[pl.ds(s * RPT, RPT)],
                        out_hbm.at[c, pl.ds(s * RPT, RPT)])

    return k(msg, idx3, zinit)


def _sc_count(idx3, ones, zinit):
    @functools.partial(
        pl.kernel,
        out_type=jax.ShapeDtypeStruct((2, NROW, H), jnp.float32),
        mesh=_sc_mesh(),
        compiler_params=pltpu.CompilerParams(use_tc_tiling_on_sc=False),
        scratch_types=[
            pltpu.VMEM_SHARED((NROW, H), jnp.float32),
            pltpu.VMEM((NCH, CHUNK), jnp.int32),
            pltpu.VMEM((CHUNK, H), jnp.float32),
        ],
    )
    def k(idx_hbm, ones_hbm, z_hbm, out_hbm, aggr_sh, idx_v, buf_v):
        c = lax.axis_index("c")
        s = lax.axis_index("s")
        wid = s * 2 + c
        pltpu.sync_copy(z_hbm.at[pl.ds(s * RPT, RPT)],
                        aggr_sh.at[pl.ds(s * RPT, RPT)])
        pltpu.sync_copy(ones_hbm, buf_v)
        pltpu.sync_copy(idx_hbm.at[wid], idx_v)
        plsc.subcore_barrier()

        @pl.loop(0, NCH)
        def _(j):
            pltpu.sync_copy(buf_v, aggr_sh.at[idx_v.at[j]], add=True)

        plsc.subcore_barrier()
        pltpu.sync_copy(aggr_sh.at[pl.ds(s * RPT, RPT)],
                        out_hbm.at[c, pl.ds(s * RPT, RPT)])

    return k(idx3, ones, zinit)


def kernel(x, edge_index, edge_attr, W0, b0, We1, be1, We2, be2,
           Wroot, broot, gamma, beta, Wy, by):
    pad = EP - E
    src_p = jnp.concatenate(
        [edge_index[0], jnp.zeros((pad,), jnp.int32)]).reshape(NW, NCH, CHUNK)
    dst_p = jnp.concatenate(
        [edge_index[1], jnp.full((pad,), SAC, jnp.int32)]).reshape(NW, NCH, CHUNK)
    zinit = jnp.zeros((NROW, H), jnp.float32)
    ones = jnp.ones((CHUNK, H), jnp.float32)
    We2b = We2.astype(jnp.bfloat16)

    lane = jnp.arange(HH, dtype=jnp.int32)
    h_of_lane = 4 * (lane // 128) + (lane % 128) // H
    R = (h_of_lane[None, :] == jnp.arange(W, dtype=jnp.int32)[:, None]
         ).astype(jnp.float32)

    zc = jnp.zeros((D_IN, W - H), jnp.float32)
    W0p = jnp.concatenate([W0, zc], axis=1)
    b0p = jnp.concatenate([b0, jnp.zeros((W - H,), jnp.float32)]).reshape(1, W)
    Wrootp = jnp.zeros((W, W), jnp.float32).at[:H, :H].set(Wroot)
    brootp = jnp.concatenate(
        [broot, jnp.zeros((W - H,), jnp.float32)]).reshape(1, W)
    gammap = jnp.concatenate(
        [gamma, jnp.zeros((W - H,), jnp.float32)]).reshape(1, W)
    betap = jnp.concatenate(
        [beta, jnp.zeros((W - H,), jnp.float32)]).reshape(1, W)
    Wy_p = jnp.zeros((W, 8), jnp.float32).at[:H, :Wy.shape[1]].set(Wy)
    by_p = jnp.concatenate(
        [by, jnp.zeros((8 - by.shape[0],), jnp.float32)]).reshape(1, 8)

    out = _lin0(x, W0p, b0p)
    theta = _theta(edge_attr, We1, be1, We2b, be2)
    cnt = _sc_count(dst_p, ones, zinit)
    for _ in range(3):
        gathered = _sc_gather(out, src_p)
        msg = _matvec(theta, gathered, R)
        parts = _sc_scatter(msg, dst_p, zinit)
        out = _root(out, parts, cnt, Wrootp, brootp)
    y = _head(out, gammap, betap, Wy_p, by_p)
    return y[:, :2]


# speedup vs baseline: 2.4232x; 1.0597x over previous
import functools

import jax
import jax.numpy as jnp
from jax import lax
from jax.experimental import pallas as pl
from jax.experimental.pallas import tpu as pltpu
from jax.experimental.pallas import tpu_sc as plsc

N = 10000
E = 160000
H = 32
D_IN = 128
D_EDGE = 16
E_HID = 128
HH = H * H
W = 128

NW = 32
CHUNK = 128
NCH = 40
EPT = NCH * CHUNK
EP = NW * EPT
NROW = 10240
SAC = N
NSUB = 16
RPT = NROW // NSUB

BE = 512
BN = 1000


def _sc_mesh():
    return plsc.VectorSubcoreMesh(core_axis_name="c", subcore_axis_name="s")


def _lin0_body(x_ref, w_ref, b_ref, o_ref):
    o_ref[...] = jnp.maximum(x_ref[...] @ w_ref[...] + b_ref[...], 0.0)


def _lin0(x, W0p, b0p):
    return pl.pallas_call(
        _lin0_body,
        grid=(N // BN,),
        in_specs=[
            pl.BlockSpec((BN, D_IN), lambda i: (i, 0)),
            pl.BlockSpec((D_IN, W), lambda i: (0, 0)),
            pl.BlockSpec((1, W), lambda i: (0, 0)),
        ],
        out_specs=pl.BlockSpec((BN, W), lambda i: (i, 0)),
        out_shape=jax.ShapeDtypeStruct((N, W), jnp.float32),
    )(x, W0p, b0p)


def _fold_msg(prod):
    acc = prod[:, 0:128]
    for q in range(1, 8):
        acc = acc + prod[:, 128 * q:128 * (q + 1)]
    m64 = acc[:, 0:64] + acc[:, 64:128]
    m32 = m64[:, 0:H] + m64[:, H:2 * H]
    return jnp.concatenate(
        [m32, jnp.zeros((m32.shape[0], W - H), jnp.float32)], axis=1)


def _theta_mv_body(ea_ref, w1_ref, b1_ref, w2_ref, b2_ref, g_ref, r_ref,
                   th_ref, m_ref):
    h = jnp.maximum(ea_ref[...] @ w1_ref[...] + b1_ref[...], 0.0)
    hb = h.astype(jnp.bfloat16)
    th = jnp.dot(hb, w2_ref[...], preferred_element_type=jnp.float32)
    th = th + b2_ref[...]
    th_ref[...] = th.astype(jnp.bfloat16)
    grep = jnp.dot(g_ref[...], r_ref[...], preferred_element_type=jnp.float32)
    m_ref[...] = _fold_msg(grep * th)


def _theta_mv(ea, We1, be1, We2b, be2, gathered, R):
    return pl.pallas_call(
        _theta_mv_body,
        grid=((E + BE - 1) // BE,),
        in_specs=[
            pl.BlockSpec((BE, D_EDGE), lambda i: (i, 0)),
            pl.BlockSpec((D_EDGE, E_HID), lambda i: (0, 0)),
            pl.BlockSpec((1, E_HID), lambda i: (0, 0)),
            pl.BlockSpec((E_HID, HH), lambda i: (0, 0)),
            pl.BlockSpec((1, HH), lambda i: (0, 0)),
            pl.BlockSpec((BE, W), lambda i: (i, 0)),
            pl.BlockSpec((W, HH), lambda i: (0, 0)),
        ],
        out_specs=[
            pl.BlockSpec((BE, HH), lambda i: (i, 0)),
            pl.BlockSpec((BE, W), lambda i: (i, 0)),
        ],
        out_shape=[
            jax.ShapeDtypeStruct((EP, HH), jnp.bfloat16),
            jax.ShapeDtypeStruct((EP, W), jnp.float32),
        ],
    )(ea, We1, be1.reshape(1, E_HID), We2b, be2.reshape(1, HH), gathered, R)


def _matvec_body(th_ref, g_ref, r_ref, o_ref):
    grep = jnp.dot(g_ref[...], r_ref[...], preferred_element_type=jnp.float32)
    o_ref[...] = _fold_msg(grep * th_ref[...].astype(jnp.float32))


def _matvec(theta, gathered, R):
    return pl.pallas_call(
        _matvec_body,
        grid=(EP // BE,),
        in_specs=[
            pl.BlockSpec((BE, HH), lambda i: (i, 0)),
            pl.BlockSpec((BE, W), lambda i: (i, 0)),
            pl.BlockSpec((W, HH), lambda i: (0, 0)),
        ],
        out_specs=pl.BlockSpec((BE, W), lambda i: (i, 0)),
        out_shape=jax.ShapeDtypeStruct((EP, W), jnp.float32),
    )(theta, gathered, R)


def _root_body(o_ref, p_ref, c_ref, w_ref, b_ref, out_ref):
    p = p_ref[...]
    cnt = c_ref[...]
    agg = p[0] + p[1]
    inv = 1.0 / jnp.maximum(cnt[0] + cnt[1], 1.0)
    aggw = jnp.concatenate(
        [agg * inv, jnp.zeros((agg.shape[0], W - H), jnp.float32)], axis=1)
    out_ref[...] = jnp.maximum(
        o_ref[...] @ w_ref[...] + aggw + b_ref[...], 0.0)


def _root(out, parts, cnt, Wrootp, brootp):
    return pl.pallas_call(
        _root_body,
        grid=(N // BN,),
        in_specs=[
            pl.BlockSpec((BN, W), lambda i: (i, 0)),
            pl.BlockSpec((2, BN, H), lambda i: (0, i, 0)),
            pl.BlockSpec((2, BN, H), lambda i: (0, i, 0)),
            pl.BlockSpec((W, W), lambda i: (0, 0)),
            pl.BlockSpec((1, W), lambda i: (0, 0)),
        ],
        out_specs=pl.BlockSpec((BN, W), lambda i: (i, 0)),
        out_shape=jax.ShapeDtypeStruct((N, W), jnp.float32),
    )(out, parts, cnt, Wrootp, brootp)


def _head_body(o_ref, g_ref, be_ref, wy_ref, by_ref, y_ref):
    scale = 1.0 / jnp.sqrt(jnp.float32(1.0 + 1e-5))
    ybn = o_ref[...] * (g_ref[...] * scale) + be_ref[...]
    logits = ybn @ wy_ref[...] + by_ref[...]
    y_ref[...] = 1.0 / (1.0 + jnp.exp(-logits))


def _head(out, gammap, betap, Wy_p, by_p):
    return pl.pallas_call(
        _head_body,
        grid=(N // BN,),
        in_specs=[
            pl.BlockSpec((BN, W), lambda i: (i, 0)),
            pl.BlockSpec((1, W), lambda i: (0, 0)),
            pl.BlockSpec((1, W), lambda i: (0, 0)),
            pl.BlockSpec((W, 8), lambda i: (0, 0)),
            pl.BlockSpec((1, 8), lambda i: (0, 0)),
        ],
        out_specs=pl.BlockSpec((BN, 8), lambda i: (i, 0)),
        out_shape=jax.ShapeDtypeStruct((N, 8), jnp.float32),
    )(out, gammap, betap, Wy_p, by_p)


def _sc_gather(nodes, idx3):
    @functools.partial(
        pl.kernel,
        out_type=jax.ShapeDtypeStruct((EP, W), jnp.float32),
        mesh=_sc_mesh(),
        scratch_types=[
            pltpu.VMEM((NCH, CHUNK), jnp.int32),
            pltpu.VMEM((CHUNK, W), jnp.float32),
            pltpu.VMEM((CHUNK, W), jnp.float32),
            pltpu.SemaphoreType.DMA,
            pltpu.SemaphoreType.DMA,
        ],
    )
    def k(nodes_hbm, idx_hbm, out_hbm, idx_v, buf_a, buf_b, sem_a, sem_b):
        c = lax.axis_index("c")
        s = lax.axis_index("s")
        wid = s * 2 + c
        pltpu.sync_copy(idx_hbm.at[wid], idx_v)
        base = wid * EPT

        @pl.loop(0, NCH, step=2)
        def _(j):
            da = pltpu.async_copy(nodes_hbm.at[idx_v.at[j]], buf_a, sem_a)
            db = pltpu.async_copy(nodes_hbm.at[idx_v.at[j + 1]], buf_b, sem_b)
            da.wait()
            pltpu.sync_copy(buf_a, out_hbm.at[pl.ds(base + j * CHUNK, CHUNK)])
            db.wait()
            pltpu.sync_copy(buf_b,
                            out_hbm.at[pl.ds(base + (j + 1) * CHUNK, CHUNK)])

    return k(nodes, idx3)


def _sc_scatter(msg, idx3, zinit):
    @functools.partial(
        pl.kernel,
        out_type=jax.ShapeDtypeStruct((2, NROW, H), jnp.float32),
        mesh=_sc_mesh(),
        compiler_params=pltpu.CompilerParams(use_tc_tiling_on_sc=False),
        scratch_types=[
            pltpu.VMEM_SHARED((NROW, H), jnp.float32),
            pltpu.VMEM((NCH, CHUNK), jnp.int32),
            pltpu.VMEM((CHUNK, W), jnp.float32),
            pltpu.VMEM((CHUNK, W), jnp.float32),
            pltpu.VMEM((CHUNK, H), jnp.float32),
            pltpu.VMEM((CHUNK, H), jnp.float32),
            pltpu.SemaphoreType.DMA,
            pltpu.SemaphoreType.DMA,
        ],
    )
    def k(msg_hbm, idx_hbm, z_hbm, out_hbm, aggr_sh, idx_v, buf_a, buf_b,
          nar_a, nar_b, sem_a, sem_b):
        c = lax.axis_index("c")
        s = lax.axis_index("s")
        wid = s * 2 + c
        pltpu.sync_copy(z_hbm.at[pl.ds(s * RPT, RPT)],
                        aggr_sh.at[pl.ds(s * RPT, RPT)])
        pltpu.sync_copy(idx_hbm.at[wid], idx_v)
        plsc.subcore_barrier()
        base = wid * EPT

        @pl.loop(0, NCH, step=2)
        def _(j):
            da = pltpu.async_copy(
                msg_hbm.at[pl.ds(base + j * CHUNK, CHUNK)], buf_a, sem_a)
            db = pltpu.async_copy(
                msg_hbm.at[pl.ds(base + (j + 1) * CHUNK, CHUNK)], buf_b, sem_b)
            da.wait()
            for r in range(CHUNK):
                nar_a[r, pl.ds(0, 16)] = buf_a[r, pl.ds(0, 16)]
                nar_a[r, pl.ds(16, 16)] = buf_a[r, pl.ds(16, 16)]
            pltpu.sync_copy(nar_a, aggr_sh.at[idx_v.at[j]], add=True)
            db.wait()
            for r in range(CHUNK):
                nar_b[r, pl.ds(0, 16)] = buf_b[r, pl.ds(0, 16)]
                nar_b[r, pl.ds(16, 16)] = buf_b[r, pl.ds(16, 16)]
            pltpu.sync_copy(nar_b, aggr_sh.at[idx_v.at[j + 1]], add=True)

        plsc.subcore_barrier()
        pltpu.sync_copy(aggr_sh.at[pl.ds(s * RPT, RPT)],
                        out_hbm.at[c, pl.ds(s * RPT, RPT)])

    return k(msg, idx3, zinit)


def _sc_count(idx3, ones, zinit):
    @functools.partial(
        pl.kernel,
        out_type=jax.ShapeDtypeStruct((2, NROW, H), jnp.float32),
        mesh=_sc_mesh(),
        compiler_params=pltpu.CompilerParams(use_tc_tiling_on_sc=False),
        scratch_types=[
            pltpu.VMEM_SHARED((NROW, H), jnp.float32),
            pltpu.VMEM((NCH, CHUNK), jnp.int32),
            pltpu.VMEM((CHUNK, H), jnp.float32),
        ],
    )
    def k(idx_hbm, ones_hbm, z_hbm, out_hbm, aggr_sh, idx_v, buf_v):
        c = lax.axis_index("c")
        s = lax.axis_index("s")
        wid = s * 2 + c
        pltpu.sync_copy(z_hbm.at[pl.ds(s * RPT, RPT)],
                        aggr_sh.at[pl.ds(s * RPT, RPT)])
        pltpu.sync_copy(ones_hbm, buf_v)
        pltpu.sync_copy(idx_hbm.at[wid], idx_v)
        plsc.subcore_barrier()

        @pl.loop(0, NCH)
        def _(j):
            pltpu.sync_copy(buf_v, aggr_sh.at[idx_v.at[j]], add=True)

        plsc.subcore_barrier()
        pltpu.sync_copy(aggr_sh.at[pl.ds(s * RPT, RPT)],
                        out_hbm.at[c, pl.ds(s * RPT, RPT)])

    return k(idx3, ones, zinit)


def kernel(x, edge_index, edge_attr, W0, b0, We1, be1, We2, be2,
           Wroot, broot, gamma, beta, Wy, by):
    pad = EP - E
    src_p = jnp.concatenate(
        [edge_index[0], jnp.zeros((pad,), jnp.int32)]).reshape(NW, NCH, CHUNK)
    sac = SAC + (jnp.arange(pad, dtype=jnp.int32) % (NROW - SAC))
    dst_p = jnp.concatenate(
        [edge_index[1], sac]).reshape(NW, NCH, CHUNK)
    zinit = jnp.zeros((NROW, H), jnp.float32)
    ones = jnp.ones((CHUNK, H), jnp.float32)
    We2b = We2.astype(jnp.bfloat16)

    lane = jnp.arange(HH, dtype=jnp.int32)
    h_of_lane = 4 * (lane // 128) + (lane % 128) // H
    R = (h_of_lane[None, :] == jnp.arange(W, dtype=jnp.int32)[:, None]
         ).astype(jnp.float32)

    zc = jnp.zeros((D_IN, W - H), jnp.float32)
    W0p = jnp.concatenate([W0, zc], axis=1)
    b0p = jnp.concatenate([b0, jnp.zeros((W - H,), jnp.float32)]).reshape(1, W)
    Wrootp = jnp.zeros((W, W), jnp.float32).at[:H, :H].set(Wroot)
    brootp = jnp.concatenate(
        [broot, jnp.zeros((W - H,), jnp.float32)]).reshape(1, W)
    gammap = jnp.concatenate(
        [gamma, jnp.zeros((W - H,), jnp.float32)]).reshape(1, W)
    betap = jnp.concatenate(
        [beta, jnp.zeros((W - H,), jnp.float32)]).reshape(1, W)
    Wy_p = jnp.zeros((W, 8), jnp.float32).at[:H, :Wy.shape[1]].set(Wy)
    by_p = jnp.concatenate(
        [by, jnp.zeros((8 - by.shape[0],), jnp.float32)]).reshape(1, 8)

    out = _lin0(x, W0p, b0p)
    cnt = _sc_count(dst_p, ones, zinit)
    gathered = _sc_gather(out, src_p)
    theta, msg = _theta_mv(edge_attr, We1, be1, We2b, be2, gathered, R)
    parts = _sc_scatter(msg, dst_p, zinit)
    out = _root(out, parts, cnt, Wrootp, brootp)
    for _ in range(2):
        gathered = _sc_gather(out, src_p)
        msg = _matvec(theta, gathered, R)
        parts = _sc_scatter(msg, dst_p, zinit)
        out = _root(out, parts, cnt, Wrootp, brootp)
    y = _head(out, gammap, betap, Wy_p, by_p)
    return y[:, :2]
